# baseline (speedup 1.0000x reference)
import functools

import jax
import jax.numpy as jnp
from jax import lax
from jax.experimental import pallas as pl
from jax.experimental.pallas import tpu as pltpu
from jax.experimental.pallas import tpu_sc as plsc

VOCAB = 1000
EMBED = 128
BATCH = 4096
SEQ = 200
N = BATCH * SEQ

NC = 2
NS = 16
NW = NC * NS
B_PER_W = N // NW
CHUNK = 128
NSETS = 5
BODY = NSETS * CHUNK
NB = B_PER_W // BODY
W_BLK = 64


@functools.partial(
    pl.kernel,
    out_type=jax.ShapeDtypeStruct((N, EMBED), jnp.float32),
    mesh=plsc.VectorSubcoreMesh(core_axis_name="c", subcore_axis_name="s"),
    scratch_types=(
        [pltpu.VMEM((B_PER_W,), jnp.int32)]
        + [pltpu.VMEM_SHARED((VOCAB, EMBED), jnp.float32)]
        + [pltpu.VMEM((CHUNK, EMBED), jnp.float32) for _ in range(NSETS)]
        + [pltpu.SemaphoreType.DMA for _ in range(2 * NSETS)]
    ),
)
def _embed_lookup(x_hbm, w_hbm, out_hbm, idx_v, w_sh, *bufs_and_sems):
    rows = bufs_and_sems[:NSETS]
    gsem = bufs_and_sems[NSETS:2 * NSETS]
    ssem = bufs_and_sems[2 * NSETS:3 * NSETS]

    sid = lax.axis_index("s")
    wid = sid * NC + lax.axis_index("c")
    base = wid * B_PER_W

    @pl.when(sid < NS - 1)
    def _():
        pltpu.sync_copy(
            w_hbm.at[pl.ds(sid * W_BLK, W_BLK)],
            w_sh.at[pl.ds(sid * W_BLK, W_BLK)],
        )

    @pl.when(sid == NS - 1)
    def _():
        pltpu.sync_copy(
            w_hbm.at[pl.ds((NS - 1) * W_BLK, VOCAB - (NS - 1) * W_BLK)],
            w_sh.at[pl.ds((NS - 1) * W_BLK, VOCAB - (NS - 1) * W_BLK)],
        )

    pltpu.sync_copy(x_hbm.at[pl.ds(base, B_PER_W)], idx_v)
    plsc.subcore_barrier()

    def idx_slice(local_off):
        return idx_v.at[pl.ds(local_off, CHUNK)]

    def step(i, carry):
        loc = i * BODY
        off = base + loc

        for s in range(NSETS):
            @pl.when(i > 0)
            def _():
                pltpu.make_async_copy(
                    rows[s], out_hbm.at[pl.ds(off, CHUNK)], ssem[s]
                ).wait()
            pltpu.async_copy(
                w_sh.at[idx_slice(loc + s * CHUNK)], rows[s], gsem[s]
            )

        for s in range(NSETS):
            pltpu.make_async_copy(
                w_sh.at[idx_slice(loc + s * CHUNK)], rows[s], gsem[s]
            ).wait()
            pltpu.async_copy(
                rows[s], out_hbm.at[pl.ds(off + s * CHUNK, CHUNK)], ssem[s]
            )
        return carry

    lax.fori_loop(0, NB, step, 0)

    for s in range(NSETS):
        pltpu.make_async_copy(
            rows[s], out_hbm.at[pl.ds(base, CHUNK)], ssem[s]
        ).wait()


def kernel(x, weight):
    xf = x.reshape(N).astype(jnp.int32)
    out = _embed_lookup(xf, weight)
    return out.reshape(BATCH, SEQ, EMBED)
